# Initial kernel scaffold; baseline (speedup 1.0000x reference)
import functools

import jax
import jax.numpy as jnp
from jax import lax
from jax.experimental import pallas as pl
from jax.experimental.pallas import tpu as pltpu
from jax.experimental.pallas import tpu_sc as plsc

BATCH = 16384
SEQ = 50
DIM = 64
NUM_ROWS = BATCH * SEQ
NUM_CORES = 2
NUM_SUBCORES = 16
NW = NUM_CORES * NUM_SUBCORES
ROWS_PER_W = NUM_ROWS // NW
CHUNK = 128
N_CHUNKS = ROWS_PER_W // CHUNK

_mesh = plsc.VectorSubcoreMesh(core_axis_name="c", subcore_axis_name="s")


@functools.partial(
    pl.kernel,
    mesh=_mesh,
    out_type=jax.ShapeDtypeStruct((NUM_ROWS, DIM), jnp.float32),
    scratch_types=[
        pltpu.VMEM((N_CHUNKS, CHUNK), jnp.int32),
        pltpu.VMEM((CHUNK, DIM), jnp.float32),
        pltpu.SemaphoreType.DMA,
    ],
)
def _gather_kernel(table_hbm, idx_hbm, out_hbm, idx_v, buf, gsem):
    wid = lax.axis_index("s") * NUM_CORES + lax.axis_index("c")
    base = wid * ROWS_PER_W
    pltpu.sync_copy(idx_hbm.at[wid], idx_v)

    def body(j, carry):
        pltpu.async_copy(table_hbm.at[idx_v.at[j]], buf, gsem).wait()
        pltpu.sync_copy(buf, out_hbm.at[pl.ds(base + j * CHUNK, CHUNK)])
        return carry

    lax.fori_loop(0, N_CHUNKS, body, 0)


def kernel(x, weight):
    idx = x.astype(jnp.int32).reshape(NW, N_CHUNKS, CHUNK)
    out = _gather_kernel(weight, idx)
    return out.reshape(BATCH, SEQ, DIM)


# speedup vs baseline: 1.6852x; 1.6852x over previous
import functools

import jax
import jax.numpy as jnp
from jax import lax
from jax.experimental import pallas as pl
from jax.experimental.pallas import tpu as pltpu
from jax.experimental.pallas import tpu_sc as plsc

BATCH = 16384
SEQ = 50
DIM = 64
NUM_ROWS = BATCH * SEQ
NUM_CORES = 2
NUM_SUBCORES = 16
NW = NUM_CORES * NUM_SUBCORES
ROWS_PER_W = NUM_ROWS // NW
CHUNK = 128
N_CHUNKS = ROWS_PER_W // CHUNK

_mesh = plsc.VectorSubcoreMesh(core_axis_name="c", subcore_axis_name="s")


@functools.partial(
    pl.kernel,
    mesh=_mesh,
    out_type=jax.ShapeDtypeStruct((NUM_ROWS, DIM), jnp.float32),
    scratch_types=[
        pltpu.VMEM((N_CHUNKS, CHUNK), jnp.int32),
        pltpu.VMEM((CHUNK, DIM), jnp.float32),
        pltpu.SemaphoreType.DMA,
    ],
    compiler_params=pltpu.CompilerParams(use_tc_tiling_on_sc=False),
)
def _gather_kernel(table_hbm, idx_hbm, out_hbm, idx_v, buf, gsem):
    wid = lax.axis_index("s") * NUM_CORES + lax.axis_index("c")
    base = wid * ROWS_PER_W
    pltpu.sync_copy(idx_hbm.at[wid], idx_v)

    def body(j, carry):
        pltpu.async_copy(table_hbm.at[idx_v.at[j]], buf, gsem).wait()
        pltpu.sync_copy(buf, out_hbm.at[pl.ds(base + j * CHUNK, CHUNK)])
        return carry

    lax.fori_loop(0, N_CHUNKS, body, 0)


def kernel(x, weight):
    idx = x.astype(jnp.int32).reshape(NW, N_CHUNKS, CHUNK)
    out = _gather_kernel(weight, idx)
    return out.reshape(BATCH, SEQ, DIM)


# speedup vs baseline: 1.8687x; 1.1089x over previous
import functools

import jax
import jax.numpy as jnp
from jax import lax
from jax.experimental import pallas as pl
from jax.experimental.pallas import tpu as pltpu
from jax.experimental.pallas import tpu_sc as plsc

BATCH = 16384
SEQ = 50
DIM = 64
NUM_ROWS = BATCH * SEQ
NUM_CORES = 2
NUM_SUBCORES = 16
NW = NUM_CORES * NUM_SUBCORES
ROWS_PER_W = NUM_ROWS // NW
CHUNK = 128
N_CHUNKS = ROWS_PER_W // CHUNK
K = 4
T = N_CHUNKS // K

_mesh = plsc.VectorSubcoreMesh(core_axis_name="c", subcore_axis_name="s")


@functools.partial(
    pl.kernel,
    mesh=_mesh,
    out_type=jax.ShapeDtypeStruct((NUM_ROWS, DIM), jnp.float32),
    scratch_types=[
        pltpu.VMEM((N_CHUNKS, CHUNK), jnp.int32),
        pltpu.VMEM((2, K, CHUNK, DIM), jnp.float32),
        pltpu.SemaphoreType.DMA,
        pltpu.SemaphoreType.DMA,
        pltpu.SemaphoreType.DMA,
        pltpu.SemaphoreType.DMA,
    ],
    compiler_params=pltpu.CompilerParams(use_tc_tiling_on_sc=False),
)
def _gather_kernel(table_hbm, idx_hbm, out_hbm, idx_v, bufs, g0, g1, s0, s1):
    wid = lax.axis_index("s") * NUM_CORES + lax.axis_index("c")
    base = wid * ROWS_PER_W
    pltpu.sync_copy(idx_hbm.at[wid], idx_v)

    gsems = (g0, g1)
    ssems = (s0, s1)

    def fire_gathers(c, t):
        for i in range(K):
            pltpu.async_copy(
                table_hbm.at[idx_v.at[t * K + i]], bufs.at[c, i], gsems[c]
            )

    def wait_gathers(c):
        for i in range(K):
            pltpu.make_async_copy(
                table_hbm.at[pl.ds(0, CHUNK)], bufs.at[c, i], gsems[c]
            ).wait()

    def fire_stores(c, t):
        for i in range(K):
            pltpu.async_copy(
                bufs.at[c, i],
                out_hbm.at[pl.ds(base + (t * K + i) * CHUNK, CHUNK)],
                ssems[c],
            )

    def wait_stores(c):
        for i in range(K):
            pltpu.make_async_copy(
                bufs.at[c, i], out_hbm.at[pl.ds(0, CHUNK)], ssems[c]
            ).wait()

    fire_gathers(0, 0)
    wait_gathers(0)
    fire_gathers(1, 1)
    fire_stores(0, 0)

    def body(t2, carry):
        t1 = 2 * t2 + 1
        wait_gathers(1)
        wait_stores(0)
        fire_gathers(0, t1 + 1)
        fire_stores(1, t1)
        t0 = t1 + 1
        wait_gathers(0)
        wait_stores(1)
        fire_gathers(1, t0 + 1)
        fire_stores(0, t0)
        return carry

    lax.fori_loop(0, (T - 2) // 2, body, 0)

    wait_gathers(1)
    wait_stores(0)
    fire_stores(1, T - 1)
    wait_stores(1)


def kernel(x, weight):
    idx = x.astype(jnp.int32).reshape(NW, N_CHUNKS, CHUNK)
    out = _gather_kernel(weight, idx)
    return out.reshape(BATCH, SEQ, DIM)
